# baseline (speedup 1.0000x reference)
import jax
import jax.numpy as jnp
from jax import lax
from jax.experimental import pallas as pl

_KB = 128


def _prologue_kernel(w0_ref, w1_ref, w2_ref, b0_ref, b1_ref, b2_ref,
                     mt_ref, ws_ref, wh_ref, bs_ref, bh_ref,
                     wcat_ref, st_ref, ht_ref):
    L, k0 = w0_ref.shape
    k1 = w1_ref.shape[1]
    k2 = w2_ref.shape[1]
    wcat_ref[:, :k0] = w0_ref[...].astype(jnp.bfloat16)
    wcat_ref[:, k0:k0 + k1] = w1_ref[...].astype(jnp.bfloat16)
    wcat_ref[:, k0 + k1:k0 + k1 + k2] = w2_ref[...].astype(jnp.bfloat16)
    j = lax.broadcasted_iota(jnp.int32, (L, _KB), 1)
    bb = (jnp.where(j == 0, b0_ref[...], 0.0)
          + jnp.where(j == 1, b1_ref[...], 0.0)
          + jnp.where(j == 2, b2_ref[...], 0.0))
    wcat_ref[:, k0 + k1 + k2:] = bb.astype(jnp.bfloat16)

    mt = mt_ref[...]
    dn = (((1,), (1,)), ((), ()))
    st_ref[...] = (lax.dot_general(mt, ws_ref[...], dn,
                                   preferred_element_type=jnp.float32)
                   + bs_ref[...]).astype(jnp.bfloat16)
    ht_ref[...] = (lax.dot_general(mt, wh_ref[...], dn,
                                   preferred_element_type=jnp.float32)
                   + bh_ref[...]).astype(jnp.bfloat16)


def _main_kernel(x_ref, mod_ref, mode_ref, wcat_ref, st_ref, ht_ref, out_ref):
    k1 = 1024
    k2 = 512
    xb = x_ref[...].astype(jnp.bfloat16)
    BM = xb.shape[0]
    mids = mod_ref[0, 0, :][:, None]
    oh3 = (mids == lax.broadcasted_iota(
        jnp.int32, (BM, _KB), 1)).astype(jnp.bfloat16)
    y = jnp.concatenate([
        (mids == 0).astype(jnp.bfloat16) * xb,
        (mids == 1).astype(jnp.bfloat16) * xb[:, :k1],
        (mids == 2).astype(jnp.bfloat16) * xb[:, :k2],
        oh3,
    ], axis=1)
    dn = (((1,), (1,)), ((), ()))
    content = lax.dot_general(y, wcat_ref[...], dn,
                              preferred_element_type=jnp.float32)

    modes = mode_ref[0, 0, :]
    n_modes = st_ref.shape[0]
    oh = (modes[:, None] == lax.broadcasted_iota(
        jnp.int32, (BM, n_modes), 1)).astype(jnp.bfloat16)
    dn2 = (((1,), (0,)), ((), ()))
    scale = lax.dot_general(oh, st_ref[...], dn2,
                            preferred_element_type=jnp.float32)
    shift = lax.dot_general(oh, ht_ref[...], dn2,
                            preferred_element_type=jnp.float32)

    out_ref[...] = content * (1.0 + scale) + shift


def kernel(features, modality_ids, mode_ids, W0, b0, W1, b1, W2, b2,
           mode_table, Ws, bs, Wh, bh):
    B, D = features.shape
    L = W0.shape[0]
    n_modes = mode_table.shape[0]
    KT = D + W1.shape[1] + W2.shape[1] + _KB

    wcat, scale_t, shift_t = pl.pallas_call(
        _prologue_kernel,
        out_shape=(jax.ShapeDtypeStruct((L, KT), jnp.bfloat16),
                   jax.ShapeDtypeStruct((n_modes, L), jnp.bfloat16),
                   jax.ShapeDtypeStruct((n_modes, L), jnp.bfloat16)),
    )(W0, W1, W2, b0.reshape(L, 1), b1.reshape(L, 1), b2.reshape(L, 1),
      mode_table, Ws, Wh, bs.reshape(1, L), bh.reshape(1, L))

    BM = 512
    NM = B // BM
    mod3 = modality_ids.reshape(NM, 1, BM)
    mode3 = mode_ids.reshape(NM, 1, BM)

    out = pl.pallas_call(
        _main_kernel,
        grid=(NM,),
        in_specs=[
            pl.BlockSpec((BM, D), lambda i: (i, 0)),
            pl.BlockSpec((1, 1, BM), lambda i: (i, 0, 0)),
            pl.BlockSpec((1, 1, BM), lambda i: (i, 0, 0)),
            pl.BlockSpec((L, KT), lambda i: (0, 0)),
            pl.BlockSpec((n_modes, L), lambda i: (0, 0)),
            pl.BlockSpec((n_modes, L), lambda i: (0, 0)),
        ],
        out_specs=pl.BlockSpec((BM, L), lambda i: (i, 0)),
        out_shape=jax.ShapeDtypeStruct((B, L), jnp.float32),
    )(features, mod3, mode3, wcat, scale_t, shift_t)
    return out


# speedup vs baseline: 1.1149x; 1.1149x over previous
import jax
import jax.numpy as jnp
from jax import lax
from jax.experimental import pallas as pl
from jax.experimental.pallas import tpu as pltpu

_KB = 128


def _tables_kernel(mt_ref, ws_ref, wh_ref, bs_ref, bh_ref, st_ref, ht_ref):
    mt = mt_ref[...]
    dn = (((1,), (1,)), ((), ()))
    st_ref[...] = (lax.dot_general(mt, ws_ref[...], dn,
                                   preferred_element_type=jnp.float32)
                   + bs_ref[...]).astype(jnp.bfloat16)
    ht_ref[...] = (lax.dot_general(mt, wh_ref[...], dn,
                                   preferred_element_type=jnp.float32)
                   + bh_ref[...]).astype(jnp.bfloat16)


def _main_kernel(x_ref, mod_ref, mode_ref, w0_ref, w1_ref, w2_ref, bcat_ref,
                 st_ref, ht_ref, out_ref, wcat_ref):
    k0 = w0_ref.shape[1]
    k1 = w1_ref.shape[1]
    k2 = w2_ref.shape[1]

    @pl.when(pl.program_id(0) == 0)
    def _():
        wcat_ref[:, :k0] = w0_ref[...].astype(jnp.bfloat16)
        wcat_ref[:, k0:k0 + k1] = w1_ref[...].astype(jnp.bfloat16)
        wcat_ref[:, k0 + k1:k0 + k1 + k2] = w2_ref[...].astype(jnp.bfloat16)
        wcat_ref[:, k0 + k1 + k2:] = bcat_ref[...].astype(jnp.bfloat16)

    xb = x_ref[...].astype(jnp.bfloat16)
    BM = xb.shape[0]
    mids = mod_ref[0, 0, :][:, None]
    oh3 = (mids == lax.broadcasted_iota(
        jnp.int32, (BM, _KB), 1)).astype(jnp.bfloat16)
    y = jnp.concatenate([
        (mids == 0).astype(jnp.bfloat16) * xb,
        (mids == 1).astype(jnp.bfloat16) * xb[:, :k1],
        (mids == 2).astype(jnp.bfloat16) * xb[:, :k2],
        oh3,
    ], axis=1)
    dn = (((1,), (1,)), ((), ()))
    content = lax.dot_general(y, wcat_ref[...], dn,
                              preferred_element_type=jnp.float32)

    modes = mode_ref[0, 0, :]
    n_modes = st_ref.shape[0]
    oh = (modes[:, None] == lax.broadcasted_iota(
        jnp.int32, (BM, n_modes), 1)).astype(jnp.bfloat16)
    dn2 = (((1,), (0,)), ((), ()))
    scale = lax.dot_general(oh, st_ref[...], dn2,
                            preferred_element_type=jnp.float32)
    shift = lax.dot_general(oh, ht_ref[...], dn2,
                            preferred_element_type=jnp.float32)

    out_ref[...] = content * (1.0 + scale) + shift


def kernel(features, modality_ids, mode_ids, W0, b0, W1, b1, W2, b2,
           mode_table, Ws, bs, Wh, bh):
    B, D = features.shape
    L = W0.shape[0]
    n_modes = mode_table.shape[0]
    KT = D + W1.shape[1] + W2.shape[1] + _KB

    scale_t, shift_t = pl.pallas_call(
        _tables_kernel,
        out_shape=(jax.ShapeDtypeStruct((n_modes, L), jnp.bfloat16),
                   jax.ShapeDtypeStruct((n_modes, L), jnp.bfloat16)),
    )(mode_table, Ws, Wh, bs.reshape(1, L), bh.reshape(1, L))

    bcat = jnp.pad(jnp.stack([b0, b1, b2], axis=1), ((0, 0), (0, _KB - 3)))

    BM = 256
    NM = B // BM
    mod3 = modality_ids.reshape(NM, 1, BM)
    mode3 = mode_ids.reshape(NM, 1, BM)

    out = pl.pallas_call(
        _main_kernel,
        grid=(NM,),
        in_specs=[
            pl.BlockSpec((BM, D), lambda i: (i, 0)),
            pl.BlockSpec((1, 1, BM), lambda i: (i, 0, 0)),
            pl.BlockSpec((1, 1, BM), lambda i: (i, 0, 0)),
            pl.BlockSpec((L, D), lambda i: (0, 0)),
            pl.BlockSpec((L, W1.shape[1]), lambda i: (0, 0)),
            pl.BlockSpec((L, W2.shape[1]), lambda i: (0, 0)),
            pl.BlockSpec((L, _KB), lambda i: (0, 0)),
            pl.BlockSpec((n_modes, L), lambda i: (0, 0)),
            pl.BlockSpec((n_modes, L), lambda i: (0, 0)),
        ],
        out_specs=pl.BlockSpec((BM, L), lambda i: (i, 0)),
        out_shape=jax.ShapeDtypeStruct((B, L), jnp.float32),
        scratch_shapes=[pltpu.VMEM((L, KT), jnp.bfloat16)],
        compiler_params=pltpu.CompilerParams(
            dimension_semantics=("arbitrary",)),
    )(features, mod3, mode3, W0, W1, W2, bcat, scale_t, shift_t)
    return out
